# baseline (speedup 1.0000x reference)
import functools

import jax
import jax.numpy as jnp
from jax import lax
from jax.experimental import pallas as pl
from jax.experimental.pallas import tpu as pltpu
from jax.experimental.pallas import tpu_sc as plsc

N = 10000
NPAD = 10112
E = 320000
D = 128
DE = 16
NC = 2
NS = 16
NW = NC * NS
EPT = 10240
NJ = EPT // 128
CS = 128
NJS = EPT // CS
KG = 4
KS = 2
HP = NJS // 2
DA = D + DE
EPAD = NW * EPT
BE = 3200
GE = E // BE
RPS = NPAD // NS

_f32 = jnp.float32


@functools.cache
def _mesh():
  return plsc.VectorSubcoreMesh(core_axis_name="c", subcore_axis_name="s",
                                num_cores=NC, num_subcores=NS)


def _segsum_body(with_deg, *refs):
  if with_deg:
    (table, idxs, idxd, zeros,
     out, dego, idx_s_v, idx_d_v, bufs_v, ones_v, dbuf_v, acc_sh, deg_sh,
     semg, sems) = refs
  else:
    (table, idxs, idxd, zeros,
     out, idx_s_v, idx_d_v, bufs_v, acc_sh, semg, sems) = refs
  c = lax.axis_index("c")
  s = lax.axis_index("s")
  wid = c * NS + s
  pltpu.sync_copy(zeros.at[pl.ds(s * RPS, RPS)], acc_sh.at[pl.ds(s * RPS, RPS)])
  if with_deg:
    for k in range(CS // 16):
      ones_v[pl.ds(k * 16, 16)] = jnp.full((16,), 1.0, _f32)
    zv = jnp.zeros((16,), _f32)

    def zb(k, carry):
      dbuf_v[pl.ds(k * 16, 16)] = zv
      return carry

    lax.fori_loop(0, (RPS + 15) // 16, zb, 0)
    pltpu.sync_copy(dbuf_v.at[pl.ds(0, RPS)], deg_sh.at[pl.ds(s * RPS, RPS)])
  plsc.subcore_barrier()

  for half in range(2):
    pltpu.sync_copy(idxs.at[wid, pl.ds(half * HP, HP)], idx_s_v)
    pltpu.sync_copy(idxd.at[wid, pl.ds(half * HP, HP)], idx_d_v)

    def phase(p, carry):
      jb = p * KS
      dg = [pltpu.async_copy(table.at[idx_s_v.at[jb + b]], bufs_v.at[b], semg)
            for b in range(KS)]
      for d in dg:
        d.wait()
      ds = [pltpu.async_copy(bufs_v.at[b], acc_sh.at[idx_d_v.at[jb + b]],
                             sems, add=True)
            for b in range(KS)]
      if with_deg:
        ds += [pltpu.async_copy(ones_v, deg_sh.at[idx_d_v.at[jb + b]],
                                sems, add=True)
               for b in range(KS)]
      for d in ds:
        d.wait()
      return carry

    lax.fori_loop(0, HP // KS, phase, 0)
  plsc.subcore_barrier()
  pltpu.sync_copy(acc_sh.at[pl.ds(s * RPS, RPS)], out.at[c, pl.ds(s * RPS, RPS)])
  if with_deg:
    pltpu.sync_copy(deg_sh.at[pl.ds(s * RPS, RPS)], dbuf_v.at[pl.ds(0, RPS)])
    pltpu.sync_copy(dbuf_v.at[pl.ds(0, RPS)],
                    dego.at[pl.ds(c * NPAD + s * RPS, RPS)])


def _segsum_deg(table, src3, dst3, zeros):
  return pl.kernel(
      functools.partial(_segsum_body, True),
      out_type=(jax.ShapeDtypeStruct((NC, NPAD, D), _f32),
                jax.ShapeDtypeStruct((NC * NPAD,), _f32)),
      mesh=_mesh(),
      scratch_types=[
          pltpu.VMEM((HP, CS), jnp.int32),
          pltpu.VMEM((HP, CS), jnp.int32),
          pltpu.VMEM((KS, CS, D), _f32),
          pltpu.VMEM((CS,), _f32),
          pltpu.VMEM((((RPS + 15) // 16) * 16,), _f32),
          pltpu.VMEM_SHARED((NPAD, D), _f32),
          pltpu.VMEM_SHARED((NPAD,), _f32),
          pltpu.SemaphoreType.DMA,
          pltpu.SemaphoreType.DMA,
      ],
  )(table, src3, dst3, zeros)


def _segsum(table, src3, dst3, zeros):
  return pl.kernel(
      functools.partial(_segsum_body, False),
      out_type=jax.ShapeDtypeStruct((NC, NPAD, D), _f32),
      mesh=_mesh(),
      scratch_types=[
          pltpu.VMEM((HP, CS), jnp.int32),
          pltpu.VMEM((HP, CS), jnp.int32),
          pltpu.VMEM((KS, CS, D), _f32),
          pltpu.VMEM_SHARED((NPAD, D), _f32),
          pltpu.SemaphoreType.DMA,
          pltpu.SemaphoreType.DMA,
      ],
  )(table, src3, dst3, zeros)


def _gather2_body(a_t, b_t, idxs, idxt, g_out,
                  idx_s_v, idx_t_v, bufs_v, sema, semb, semw):
  c = lax.axis_index("c")
  s = lax.axis_index("s")
  wid = c * NS + s
  pltpu.sync_copy(idxs.at[wid], idx_s_v)
  pltpu.sync_copy(idxt.at[wid], idx_t_v)

  def phase(p, carry):
    jb = p * KG
    da = [pltpu.async_copy(a_t.at[idx_s_v.at[jb + b]], bufs_v.at[b], sema)
          for b in range(KG)]
    for d in da:
      d.wait()
    db = [pltpu.async_copy(b_t.at[idx_t_v.at[jb + b]], bufs_v.at[b], semb,
                           add=True)
          for b in range(KG)]
    for d in db:
      d.wait()
    dw = [pltpu.async_copy(
        bufs_v.at[b], g_out.at[pl.ds(wid * EPT + (jb + b) * 128, 128)], semw)
        for b in range(KG)]
    for d in dw:
      d.wait()
    return carry

  lax.fori_loop(0, NJ // KG, phase, 0)


def _gather2(*args):
  return pl.kernel(
      _gather2_body,
      out_type=jax.ShapeDtypeStruct((EPAD, D), _f32),
      mesh=_mesh(),
      scratch_types=[
          pltpu.VMEM((NJ, 128), jnp.int32),
          pltpu.VMEM((NJ, 128), jnp.int32),
          pltpu.VMEM((KG, 128, D), _f32),
          pltpu.SemaphoreType.DMA,
          pltpu.SemaphoreType.DMA,
          pltpu.SemaphoreType.DMA,
      ],
  )(*args)


def _dotT(x, w, hi=False):
  return lax.dot_general(x, w, (((1,), (1,)), ((), ())),
                         precision=lax.Precision.HIGHEST if hi else None,
                         preferred_element_type=_f32)


def _conv_body(with_ab, *refs):
  if with_ab:
    (parts, rinv_in, hprev, wl, bl, wr, g, bb, wma, wmb,
     a_out, b_out) = refs
  else:
    (parts, deg_in, xin, wl, bl, wr, g, bb, h_out, rinv_out) = refs
  p = parts[...]
  agg = p[0, :N, :] + p[1, :N, :]
  if with_ab:
    rinv = rinv_in[...]
    x = hprev[...]
  else:
    rinv = 1.0 / jnp.maximum(deg_in[...], 1.0)
    x = xin[...]
  t = _dotT(agg * rinv, wl[...]) + bl[...] + _dotT(x, wr[...])
  m = jnp.mean(t, axis=0, keepdims=True)
  v = jnp.mean((t - m) ** 2, axis=0, keepdims=True)
  h = jnp.maximum((t - m) * lax.rsqrt(v + 1e-5) * g[...] + bb[...], 0.0)
  if with_ab:
    a_out[...] = _dotT(h, wma[...], hi=True)
    b_out[...] = _dotT(h, wmb[...], hi=True)
  else:
    h_out[...] = h
    rinv_out[...] = rinv


def _conv1(parts, deg, x, wl, bl, wr, g, bb):
  return pl.pallas_call(
      functools.partial(_conv_body, False),
      out_shape=(jax.ShapeDtypeStruct((N, D), _f32),
                 jax.ShapeDtypeStruct((N, 1), _f32)),
  )(parts, deg, x, wl, bl, wr, g, bb)


def _conv2(parts, rinv, h1, wl, bl, wr, g, bb, wma, wmb):
  return pl.pallas_call(
      functools.partial(_conv_body, True),
      out_shape=(jax.ShapeDtypeStruct((N, D), _f32),
                 jax.ShapeDtypeStruct((N, D), _f32)),
  )(parts, rinv, h1, wl, bl, wr, g, bb, wma, wmb)


def _eeT(eat, we, bec):
  return jnp.maximum(
      lax.dot_general(we, eat, (((1,), (0,)), ((), ())),
                      precision=lax.Precision.HIGHEST,
                      preferred_element_type=_f32) + bec, 0.0)


def _eestats_body(eat, we, bec, sums):
  e = _eeT(eat[...], we[...], bec[...])
  blk = jnp.concatenate([jnp.sum(e, axis=1, keepdims=True),
                         jnp.sum(e * e, axis=1, keepdims=True)], axis=1)

  @pl.when(pl.program_id(0) == 0)
  def _():
    sums[...] = jnp.zeros_like(sums)

  sums[...] += blk


def _eestats(eat, we, bec):
  return pl.pallas_call(
      _eestats_body,
      grid=(GE,),
      in_specs=[
          pl.BlockSpec((DE, BE), lambda i: (0, i)),
          pl.BlockSpec((D, DE), lambda i: (0, 0)),
          pl.BlockSpec((D, 1), lambda i: (0, 0)),
      ],
      out_specs=pl.BlockSpec((D, 2), lambda i: (0, 0)),
      out_shape=jax.ShapeDtypeStruct((D, 2), _f32),
  )(eat, we, bec)


def _zblock(g, eat, we, bec, wc, b1):
  e = _eeT(eat, we, bec)
  c = lax.dot_general(e, wc, (((0,), (1,)), ((), ())),
                      precision=lax.Precision.HIGHEST,
                      preferred_element_type=_f32)
  return jnp.maximum(g + c + b1, 0.0)


def _zstats_body(g_in, eat, we, bec, wc, b1, sums):
  z = _zblock(g_in[...], eat[...], we[...], bec[...], wc[...], b1[...])
  blk = jnp.concatenate([jnp.sum(z, axis=0, keepdims=True),
                         jnp.sum(z * z, axis=0, keepdims=True)], axis=0)

  @pl.when(pl.program_id(0) == 0)
  def _():
    sums[...] = jnp.zeros_like(sums)

  sums[...] += blk


def _zstats(g_arr, eat, we, bec, wc, b1):
  return pl.pallas_call(
      _zstats_body,
      grid=(GE,),
      in_specs=[
          pl.BlockSpec((BE, D), lambda i: (i, 0)),
          pl.BlockSpec((DE, BE), lambda i: (0, i)),
          pl.BlockSpec((D, DE), lambda i: (0, 0)),
          pl.BlockSpec((D, 1), lambda i: (0, 0)),
          pl.BlockSpec((D, D), lambda i: (0, 0)),
          pl.BlockSpec((1, D), lambda i: (0, 0)),
      ],
      out_specs=pl.BlockSpec((2, D), lambda i: (0, 0)),
      out_shape=jax.ShapeDtypeStruct((2, D), _f32),
  )(g_arr, eat, we, bec, wc, b1)


def _head_body(g_in, eat, we, bec, wc, b1, wm2, b2, wm3, bm3, o):
  z = _zblock(g_in[...], eat[...], we[...], bec[...], wc[...], b1[...])
  z2 = jnp.maximum(_dotT(z, wm2[...]) + b2[...], 0.0)
  o[0] = lax.dot_general(wm3[...], z2, (((1,), (1,)), ((), ())),
                         preferred_element_type=_f32) + bm3[...]


def _head(g_arr, eat, we, bec, wc, b1, wm2, b2, wm3, bm3):
  return pl.pallas_call(
      _head_body,
      grid=(GE,),
      in_specs=[
          pl.BlockSpec((BE, D), lambda i: (i, 0)),
          pl.BlockSpec((DE, BE), lambda i: (0, i)),
          pl.BlockSpec((D, DE), lambda i: (0, 0)),
          pl.BlockSpec((D, 1), lambda i: (0, 0)),
          pl.BlockSpec((D, D), lambda i: (0, 0)),
          pl.BlockSpec((1, D), lambda i: (0, 0)),
          pl.BlockSpec((D // 2, D), lambda i: (0, 0)),
          pl.BlockSpec((1, D // 2), lambda i: (0, 0)),
          pl.BlockSpec((1, D // 2), lambda i: (0, 0)),
          pl.BlockSpec((1, 1), lambda i: (0, 0)),
      ],
      out_specs=pl.BlockSpec((1, 1, BE), lambda i: (i, 0, 0)),
      out_shape=jax.ShapeDtypeStruct((GE, 1, BE), _f32),
  )(g_arr, eat, we, bec, wc, b1, wm2, b2, wm3, bm3)


def kernel(x, edge_index, edge_attr, edge_source, edge_target,
           W_l1, b_l1, W_r1, g1, bb1,
           W_l2, b_l2, W_r2, g2, bb2,
           We, be, ge, bbe,
           Wm1, bm1, gm, bbm, Wm2, bm2, Wm3, bm3):
  src = edge_index[0].astype(jnp.int32)
  dst = edge_index[1].astype(jnp.int32)
  es = edge_source.astype(jnp.int32)
  et = edge_target.astype(jnp.int32)

  npad = EPAD - E
  ar = jnp.arange(npad, dtype=jnp.int32)
  padr = ar % N
  padn = N + ar % (NPAD - N)
  src3 = jnp.concatenate([src, padr]).reshape(NW, NJS, CS)
  dst3 = jnp.concatenate([dst, padn]).reshape(NW, NJS, CS)
  es3 = jnp.concatenate([es, padr]).reshape(NW, NJ, 128)
  et3 = jnp.concatenate([et, padr]).reshape(NW, NJ, 128)

  zeros = jnp.zeros((NPAD, D), _f32)

  r = lambda a: a.reshape(1, -1)

  parts1, degp = _segsum_deg(x, src3, dst3, zeros)
  deg = degp.reshape(NC, NPAD).sum(axis=0)[:N].reshape(N, 1)
  h1, rinv = _conv1(parts1, deg, x, W_l1, r(b_l1), W_r1, r(g1), r(bb1))
  parts2 = _segsum(h1, src3, dst3, zeros)
  a_t, b_t = _conv2(parts2, rinv, h1, W_l2, r(b_l2), W_r2, r(g2), r(bb2),
                    Wm1[:, :D], Wm1[:, D:2 * D])
  g_arr = _gather2(a_t, b_t, es3, et3)

  eat = edge_attr.T
  bec = be.reshape(-1, 1)
  s_e = _eestats(eat, We, bec)
  me = s_e[:, 0] / E
  ve = s_e[:, 1] / E - me * me
  se = ge * lax.rsqrt(ve + 1e-5)
  wmc = Wm1[:, 2 * D:]
  wc_eff = wmc * se[None, :]
  b1_eff = r(bm1 + (bbe - me * se) @ wmc.T)

  s_z = _zstats(g_arr, eat, We, bec, wc_eff, b1_eff)
  mz = s_z[0] / E
  vz = s_z[1] / E - mz * mz
  sz = gm * lax.rsqrt(vz + 1e-5)
  wm2_eff = Wm2 * sz[None, :]
  b2_eff = r(bm2 + (bbm - mz * sz) @ Wm2.T)

  out = _head(g_arr, eat, We, bec, wc_eff, b1_eff,
              wm2_eff, b2_eff, r(Wm3), bm3.reshape(1, 1))
  return out.reshape(E)


# speedup vs baseline: 1.3536x; 1.3536x over previous
import functools

import jax
import jax.numpy as jnp
from jax import lax
from jax.experimental import pallas as pl
from jax.experimental.pallas import tpu as pltpu
from jax.experimental.pallas import tpu_sc as plsc

N = 10000
NPAD = 10112
E = 320000
D = 128
DE = 16
NC = 2
NS = 16
NW = NC * NS
EPT = 10240
NJ = EPT // 128
CS = 128
NJS = EPT // CS
KG = 4
KS = 2
HP = NJS // 2
DA = D + DE
EPAD = NW * EPT
BE = 3200
GE = E // BE
RPS = NPAD // NS

_f32 = jnp.float32


@functools.cache
def _mesh():
  return plsc.VectorSubcoreMesh(core_axis_name="c", subcore_axis_name="s",
                                num_cores=NC, num_subcores=NS)


def _segsum_body(with_deg, *refs):
  if with_deg:
    (table, idxs, idxd, zeros,
     out, dego, idx_s_v, idx_d_v, bufs_v, ones_v, dbuf_v, acc_sh, deg_sh,
     semg, sems) = refs
  else:
    (table, idxs, idxd, zeros,
     out, idx_s_v, idx_d_v, bufs_v, acc_sh, semg, sems) = refs
  c = lax.axis_index("c")
  s = lax.axis_index("s")
  wid = c * NS + s
  pltpu.sync_copy(zeros.at[pl.ds(s * RPS, RPS)], acc_sh.at[pl.ds(s * RPS, RPS)])
  if with_deg:
    for k in range(CS // 16):
      ones_v[pl.ds(k * 16, 16)] = jnp.full((16,), 1.0, _f32)
    zv = jnp.zeros((16,), _f32)

    def zb(k, carry):
      dbuf_v[pl.ds(k * 16, 16)] = zv
      return carry

    lax.fori_loop(0, (RPS + 15) // 16, zb, 0)
    pltpu.sync_copy(dbuf_v.at[pl.ds(0, RPS)], deg_sh.at[pl.ds(s * RPS, RPS)])
  plsc.subcore_barrier()

  for half in range(2):
    pltpu.sync_copy(idxs.at[wid, pl.ds(half * HP, HP)], idx_s_v)
    pltpu.sync_copy(idxd.at[wid, pl.ds(half * HP, HP)], idx_d_v)

    def phase(p, carry):
      jb = p * KS
      dg = [pltpu.async_copy(table.at[idx_s_v.at[jb + b]], bufs_v.at[b], semg)
            for b in range(KS)]
      for d in dg:
        d.wait()
      ds = [pltpu.async_copy(bufs_v.at[b], acc_sh.at[idx_d_v.at[jb + b]],
                             sems, add=True)
            for b in range(KS)]
      if with_deg:
        ds += [pltpu.async_copy(ones_v, deg_sh.at[idx_d_v.at[jb + b]],
                                sems, add=True)
               for b in range(KS)]
      for d in ds:
        d.wait()
      return carry

    lax.fori_loop(0, HP // KS, phase, 0)
  plsc.subcore_barrier()
  pltpu.sync_copy(acc_sh.at[pl.ds(s * RPS, RPS)], out.at[c, pl.ds(s * RPS, RPS)])
  if with_deg:
    pltpu.sync_copy(deg_sh.at[pl.ds(s * RPS, RPS)], dbuf_v.at[pl.ds(0, RPS)])
    pltpu.sync_copy(dbuf_v.at[pl.ds(0, RPS)],
                    dego.at[pl.ds(c * NPAD + s * RPS, RPS)])


def _segsum_deg(table, src3, dst3, zeros):
  return pl.kernel(
      functools.partial(_segsum_body, True),
      out_type=(jax.ShapeDtypeStruct((NC, NPAD, D), _f32),
                jax.ShapeDtypeStruct((NC * NPAD,), _f32)),
      mesh=_mesh(),
      scratch_types=[
          pltpu.VMEM((HP, CS), jnp.int32),
          pltpu.VMEM((HP, CS), jnp.int32),
          pltpu.VMEM((KS, CS, D), _f32),
          pltpu.VMEM((CS,), _f32),
          pltpu.VMEM((((RPS + 15) // 16) * 16,), _f32),
          pltpu.VMEM_SHARED((NPAD, D), _f32),
          pltpu.VMEM_SHARED((NPAD,), _f32),
          pltpu.SemaphoreType.DMA,
          pltpu.SemaphoreType.DMA,
      ],
  )(table, src3, dst3, zeros)


def _segsum(table, src3, dst3, zeros):
  return pl.kernel(
      functools.partial(_segsum_body, False),
      out_type=jax.ShapeDtypeStruct((NC, NPAD, D), _f32),
      mesh=_mesh(),
      scratch_types=[
          pltpu.VMEM((HP, CS), jnp.int32),
          pltpu.VMEM((HP, CS), jnp.int32),
          pltpu.VMEM((KS, CS, D), _f32),
          pltpu.VMEM_SHARED((NPAD, D), _f32),
          pltpu.SemaphoreType.DMA,
          pltpu.SemaphoreType.DMA,
      ],
  )(table, src3, dst3, zeros)


def _gather2_body(a_t, b_t, idxs, idxt, g_out,
                  idx_s_v, idx_t_v, bufs_v, sema, semb, semw):
  c = lax.axis_index("c")
  s = lax.axis_index("s")
  wid = c * NS + s
  pltpu.sync_copy(idxs.at[wid], idx_s_v)
  pltpu.sync_copy(idxt.at[wid], idx_t_v)

  def phase(p, carry):
    jb = p * KG
    da = [pltpu.async_copy(a_t.at[idx_s_v.at[jb + b]], bufs_v.at[b], sema)
          for b in range(KG)]
    for d in da:
      d.wait()
    db = [pltpu.async_copy(b_t.at[idx_t_v.at[jb + b]], bufs_v.at[b], semb,
                           add=True)
          for b in range(KG)]
    for d in db:
      d.wait()
    dw = [pltpu.async_copy(
        bufs_v.at[b], g_out.at[pl.ds(wid * EPT + (jb + b) * 128, 128)], semw)
        for b in range(KG)]
    for d in dw:
      d.wait()
    return carry

  lax.fori_loop(0, NJ // KG, phase, 0)


def _gather2(*args):
  return pl.kernel(
      _gather2_body,
      out_type=jax.ShapeDtypeStruct((EPAD, D), _f32),
      mesh=_mesh(),
      scratch_types=[
          pltpu.VMEM((NJ, 128), jnp.int32),
          pltpu.VMEM((NJ, 128), jnp.int32),
          pltpu.VMEM((KG, 128, D), _f32),
          pltpu.SemaphoreType.DMA,
          pltpu.SemaphoreType.DMA,
          pltpu.SemaphoreType.DMA,
      ],
  )(*args)


def _dotT(x, w, hi=False):
  return lax.dot_general(x, w, (((1,), (1,)), ((), ())),
                         precision=lax.Precision.HIGHEST if hi else None,
                         preferred_element_type=_f32)


def _conv_body(with_ab, *refs):
  if with_ab:
    (parts, rinv_in, hprev, wl, bl, wr, g, bb, wma, wmb,
     a_out, b_out) = refs
  else:
    (parts, deg_in, xin, wl, bl, wr, g, bb, h_out, rinv_out) = refs
  p = parts[...]
  agg = p[0, :N, :] + p[1, :N, :]
  if with_ab:
    rinv = rinv_in[...]
    x = hprev[...]
  else:
    rinv = 1.0 / jnp.maximum(deg_in[...], 1.0)
    x = xin[...]
  t = _dotT(agg * rinv, wl[...]) + bl[...] + _dotT(x, wr[...])
  m = jnp.mean(t, axis=0, keepdims=True)
  v = jnp.mean((t - m) ** 2, axis=0, keepdims=True)
  h = jnp.maximum((t - m) * lax.rsqrt(v + 1e-5) * g[...] + bb[...], 0.0)
  if with_ab:
    a_out[...] = _dotT(h, wma[...], hi=True)
    b_out[...] = _dotT(h, wmb[...], hi=True)
  else:
    h_out[...] = h
    rinv_out[...] = rinv


def _conv1(parts, deg, x, wl, bl, wr, g, bb):
  return pl.pallas_call(
      functools.partial(_conv_body, False),
      out_shape=(jax.ShapeDtypeStruct((N, D), _f32),
                 jax.ShapeDtypeStruct((N, 1), _f32)),
  )(parts, deg, x, wl, bl, wr, g, bb)


def _conv2(parts, rinv, h1, wl, bl, wr, g, bb, wma, wmb):
  return pl.pallas_call(
      functools.partial(_conv_body, True),
      out_shape=(jax.ShapeDtypeStruct((N, D), _f32),
                 jax.ShapeDtypeStruct((N, D), _f32)),
  )(parts, rinv, h1, wl, bl, wr, g, bb, wma, wmb)


def _ee(ea, we, be):
  return jnp.maximum(_dotT(ea, we) + be, 0.0)


def _eestats_body(ea, we, be, sums):
  e = _ee(ea[...], we[...], be[...])
  blk = jnp.concatenate([jnp.sum(e, axis=0, keepdims=True),
                         jnp.sum(e * e, axis=0, keepdims=True)], axis=0)

  @pl.when(pl.program_id(0) == 0)
  def _():
    sums[...] = jnp.zeros_like(sums)

  sums[...] += blk


def _eestats(ea, we, be):
  return pl.pallas_call(
      _eestats_body,
      grid=(GE,),
      in_specs=[
          pl.BlockSpec((BE, DE), lambda i: (i, 0)),
          pl.BlockSpec((D, DE), lambda i: (0, 0)),
          pl.BlockSpec((1, D), lambda i: (0, 0)),
      ],
      out_specs=pl.BlockSpec((2, D), lambda i: (0, 0)),
      out_shape=jax.ShapeDtypeStruct((2, D), _f32),
  )(ea, we, be)


def _zblock(g, ea, we, be, wc, b1):
  e = _ee(ea, we, be)
  return jnp.maximum(g + _dotT(e, wc) + b1, 0.0)


def _zstats_body(g_in, ea, we, be, wc, b1, sums):
  z = _zblock(g_in[...], ea[...], we[...], be[...], wc[...], b1[...])
  blk = jnp.concatenate([jnp.sum(z, axis=0, keepdims=True),
                         jnp.sum(z * z, axis=0, keepdims=True)], axis=0)

  @pl.when(pl.program_id(0) == 0)
  def _():
    sums[...] = jnp.zeros_like(sums)

  sums[...] += blk


def _zstats(g_arr, ea, we, be, wc, b1):
  return pl.pallas_call(
      _zstats_body,
      grid=(GE,),
      in_specs=[
          pl.BlockSpec((BE, D), lambda i: (i, 0)),
          pl.BlockSpec((BE, DE), lambda i: (i, 0)),
          pl.BlockSpec((D, DE), lambda i: (0, 0)),
          pl.BlockSpec((1, D), lambda i: (0, 0)),
          pl.BlockSpec((D, D), lambda i: (0, 0)),
          pl.BlockSpec((1, D), lambda i: (0, 0)),
      ],
      out_specs=pl.BlockSpec((2, D), lambda i: (0, 0)),
      out_shape=jax.ShapeDtypeStruct((2, D), _f32),
  )(g_arr, ea, we, be, wc, b1)


def _head_body(g_in, ea, we, be, wc, b1, wm2, b2, wm3, bm3, o):
  z = _zblock(g_in[...], ea[...], we[...], be[...], wc[...], b1[...])
  z2 = jnp.maximum(_dotT(z, wm2[...]) + b2[...], 0.0)
  o[0] = lax.dot_general(wm3[...], z2, (((1,), (1,)), ((), ())),
                         preferred_element_type=_f32) + bm3[...]


def _head(g_arr, ea, we, be, wc, b1, wm2, b2, wm3, bm3):
  return pl.pallas_call(
      _head_body,
      grid=(GE,),
      in_specs=[
          pl.BlockSpec((BE, D), lambda i: (i, 0)),
          pl.BlockSpec((BE, DE), lambda i: (i, 0)),
          pl.BlockSpec((D, DE), lambda i: (0, 0)),
          pl.BlockSpec((1, D), lambda i: (0, 0)),
          pl.BlockSpec((D, D), lambda i: (0, 0)),
          pl.BlockSpec((1, D), lambda i: (0, 0)),
          pl.BlockSpec((D // 2, D), lambda i: (0, 0)),
          pl.BlockSpec((1, D // 2), lambda i: (0, 0)),
          pl.BlockSpec((1, D // 2), lambda i: (0, 0)),
          pl.BlockSpec((1, 1), lambda i: (0, 0)),
      ],
      out_specs=pl.BlockSpec((1, 1, BE), lambda i: (i, 0, 0)),
      out_shape=jax.ShapeDtypeStruct((GE, 1, BE), _f32),
  )(g_arr, ea, we, be, wc, b1, wm2, b2, wm3, bm3)


def kernel(x, edge_index, edge_attr, edge_source, edge_target,
           W_l1, b_l1, W_r1, g1, bb1,
           W_l2, b_l2, W_r2, g2, bb2,
           We, be, ge, bbe,
           Wm1, bm1, gm, bbm, Wm2, bm2, Wm3, bm3):
  src = edge_index[0].astype(jnp.int32)
  dst = edge_index[1].astype(jnp.int32)
  es = edge_source.astype(jnp.int32)
  et = edge_target.astype(jnp.int32)

  npad = EPAD - E
  ar = jnp.arange(npad, dtype=jnp.int32)
  padr = ar % N
  padn = N + ar % (NPAD - N)
  src3 = jnp.concatenate([src, padr]).reshape(NW, NJS, CS)
  dst3 = jnp.concatenate([dst, padn]).reshape(NW, NJS, CS)
  es3 = jnp.concatenate([es, padr]).reshape(NW, NJ, 128)
  et3 = jnp.concatenate([et, padr]).reshape(NW, NJ, 128)

  zeros = jnp.zeros((NPAD, D), _f32)

  r = lambda a: a.reshape(1, -1)

  parts1, degp = _segsum_deg(x, src3, dst3, zeros)
  deg = degp.reshape(NC, NPAD).sum(axis=0)[:N].reshape(N, 1)
  h1, rinv = _conv1(parts1, deg, x, W_l1, r(b_l1), W_r1, r(g1), r(bb1))
  parts2 = _segsum(h1, src3, dst3, zeros)
  a_t, b_t = _conv2(parts2, rinv, h1, W_l2, r(b_l2), W_r2, r(g2), r(bb2),
                    Wm1[:, :D], Wm1[:, D:2 * D])
  g_arr = _gather2(a_t, b_t, es3, et3)

  s_e = _eestats(edge_attr, We, r(be))
  me = s_e[0] / E
  ve = s_e[1] / E - me * me
  se = ge * lax.rsqrt(ve + 1e-5)
  wmc = Wm1[:, 2 * D:]
  wc_eff = wmc * se[None, :]
  b1_eff = r(bm1 + (bbe - me * se) @ wmc.T)

  s_z = _zstats(g_arr, edge_attr, We, r(be), wc_eff, b1_eff)
  mz = s_z[0] / E
  vz = s_z[1] / E - mz * mz
  sz = gm * lax.rsqrt(vz + 1e-5)
  wm2_eff = Wm2 * sz[None, :]
  b2_eff = r(bm2 + (bbm - mz * sz) @ Wm2.T)

  out = _head(g_arr, edge_attr, We, r(be), wc_eff, b1_eff,
              wm2_eff, b2_eff, r(Wm3), bm3.reshape(1, 1))
  return out.reshape(E)
